# Initial kernel scaffold; baseline (speedup 1.0000x reference)
import jax
import jax.numpy as jnp
from jax.experimental import pallas as pl


def kernel(vertices, feature_map, W1, b1, g1, be1, W2, b2, Wa1, ba1, ga, bea, Wa2, ba2):
    raise NotImplementedError("write your pallas kernel here")


# speedup vs baseline: 107.4806x; 107.4806x over previous
import functools

import jax
import jax.numpy as jnp
from jax import lax
from jax.experimental import pallas as pl
from jax.experimental.pallas import tpu as pltpu
from jax.experimental.pallas import tpu_sc as plsc

B = 8
N = 4096
M = 1024
K = 16
C = 64
QBLK = 128
MBLK = 256
NROW = B * M * K
NKEY = B * M
NALL = NKEY + NROW


def _fps_body(v_ref, kp_ref):
    x = v_ref[:, 0, :]
    y = v_ref[:, 1, :]
    z = v_ref[:, 2, :]
    iota = lax.broadcasted_iota(jnp.int32, (B, N), 1)
    boff = lax.broadcasted_iota(jnp.int32, (B, 1), 0) * N

    def body(i, carry):
        dists, far = carry
        oh = (iota == far).astype(jnp.float32)
        cx = jnp.sum(x * oh, axis=1, keepdims=True)
        cy = jnp.sum(y * oh, axis=1, keepdims=True)
        cz = jnp.sum(z * oh, axis=1, keepdims=True)
        fid = (far + boff).astype(jnp.float32)
        row = jnp.concatenate([cx, cy, cz, fid], axis=1)
        kp_ref[pl.ds(i, 1)] = row.reshape(1, B, 4)
        dx = x - cx
        dy = y - cy
        dz = z - cz
        d = (dx * dx + dy * dy) + dz * dz
        dists = jnp.minimum(dists, d)
        mx = jnp.max(dists, axis=1, keepdims=True)
        cand = jnp.where(dists == mx, iota, jnp.int32(N))
        far = jnp.min(cand, axis=1, keepdims=True)
        return dists, far

    dists0 = jnp.full((B, N), 1e10, dtype=jnp.float32)
    far0 = jnp.zeros((B, 1), dtype=jnp.int32)
    lax.fori_loop(0, M, body, (dists0, far0))


def _fps(vertices):
    return pl.pallas_call(
        _fps_body,
        out_shape=jax.ShapeDtypeStruct((M, B, 4), jnp.float32),
    )(vertices)


def _knn_body(v_ref, kp_ref, out_ref):
    b = pl.program_id(0) // (M // QBLK)
    v = v_ref[0]
    q = kp_ref[0][:, :3]
    qx = lax.dot_general(q, v, (((1,), (0,)), ((), ())),
                         preferred_element_type=jnp.float32)
    qn = jnp.sum(q * q, axis=1, keepdims=True)
    xn = jnp.sum(v * v, axis=0, keepdims=True)
    d = (-2.0 * qx + qn) + xn
    iota = lax.broadcasted_iota(jnp.int32, (QBLK, N), 1)
    cols = []
    for _ in range(K):
        mv = jnp.min(d, axis=1, keepdims=True)
        cand = jnp.where(d == mv, iota, jnp.int32(N))
        mi = jnp.min(cand, axis=1, keepdims=True)
        cols.append(mi)
        d = jnp.where(iota == mi, jnp.float32(jnp.inf), d)
    out_ref[0] = jnp.concatenate(cols, axis=1) + b * N


def _knn(vertices, keys):
    nq = M // QBLK
    grid = (B * nq,)
    return pl.pallas_call(
        _knn_body,
        grid=grid,
        in_specs=[
            pl.BlockSpec((1, 3, N), lambda p: (p // nq, 0, 0)),
            pl.BlockSpec((1, QBLK, 4), lambda p: (p // nq, p % nq, 0)),
        ],
        out_specs=pl.BlockSpec((1, QBLK, K), lambda p: (p, 0, 0)),
        out_shape=jax.ShapeDtypeStruct((B * nq, QBLK, K), jnp.int32),
    )(vertices, keys)


_SC_CHUNK = 128


def _sc_gather(table_f, table_v, idx):
    info = plsc.get_sparse_core_info()
    nc, ns = info.num_cores, info.num_subcores
    nw = nc * ns
    rows_per_w = NALL // nw
    nchunks = rows_per_w // _SC_CHUNK
    mesh = plsc.VectorSubcoreMesh(core_axis_name="c", subcore_axis_name="s")

    @functools.partial(
        pl.kernel,
        out_type=(
            jax.ShapeDtypeStruct((NALL, C), jnp.float32),
            jax.ShapeDtypeStruct((NALL, 16), jnp.float32),
        ),
        mesh=mesh,
        scratch_types=[
            pltpu.VMEM((_SC_CHUNK,), jnp.int32),
            pltpu.VMEM((_SC_CHUNK, C), jnp.float32),
            pltpu.VMEM((_SC_CHUNK, 16), jnp.float32),
            pltpu.SemaphoreType.DMA,
        ],
        compiler_params=pltpu.CompilerParams(use_tc_tiling_on_sc=False),
    )
    def k(tf_hbm, tv_hbm, idx_hbm, of_hbm, ov_hbm, idx_v, buf_f, buf_v, sem):
        wid = lax.axis_index("s") * nc + lax.axis_index("c")
        base = wid * rows_per_w

        def chunk(ci, carry):
            start = base + ci * _SC_CHUNK
            pltpu.sync_copy(idx_hbm.at[pl.ds(start, _SC_CHUNK)], idx_v)
            cp1 = pltpu.async_copy(tf_hbm.at[idx_v], buf_f, sem)
            cp2 = pltpu.async_copy(tv_hbm.at[idx_v], buf_v, sem)
            cp1.wait()
            cp2.wait()
            pltpu.sync_copy(buf_f, of_hbm.at[pl.ds(start, _SC_CHUNK)])
            pltpu.sync_copy(buf_v, ov_hbm.at[pl.ds(start, _SC_CHUNK)])
            return carry

        lax.fori_loop(0, nchunks, chunk, 0)

    return k(table_f, table_v, idx)


def _p3a_body(gv_ref, kv_ref, w1_ref, b1_ref, pe1_ref, st_ref):
    p = pl.program_id(0)
    gpt = gv_ref[...]
    key = kv_ref[...]
    pos = key.reshape(MBLK, 1, 16) - gpt.reshape(MBLK, K, 16)
    pos = pos.reshape(MBLK * K, 16)
    pe1 = lax.dot_general(pos, w1_ref[...], (((1,), (0,)), ((), ())),
                          preferred_element_type=jnp.float32) + b1_ref[...]
    pe1_ref[...] = pe1

    @pl.when(p == 0)
    def _():
        st_ref[...] = jnp.zeros((8, C), jnp.float32)

    st_ref[0:1, :] += jnp.sum(pe1, axis=0, keepdims=True)
    st_ref[1:2, :] += jnp.sum(pe1 * pe1, axis=0, keepdims=True)


def _pass3a(group_v, key_v, w1t, b1):
    grid = (B * M // MBLK,)
    r = MBLK * K
    return pl.pallas_call(
        _p3a_body,
        grid=grid,
        in_specs=[
            pl.BlockSpec((r, 16), lambda p: (p + NKEY // r, 0)),
            pl.BlockSpec((MBLK, 16), lambda p: (p, 0)),
            pl.BlockSpec((16, C), lambda p: (0, 0)),
            pl.BlockSpec((1, C), lambda p: (0, 0)),
        ],
        out_specs=[
            pl.BlockSpec((r, C), lambda p: (p, 0)),
            pl.BlockSpec((8, C), lambda p: (0, 0)),
        ],
        out_shape=[
            jax.ShapeDtypeStruct((NROW, C), jnp.float32),
            jax.ShapeDtypeStruct((8, C), jnp.float32),
        ],
    )(group_v, key_v, w1t, b1)


def _p3b_body(pe1_ref, gf_ref, kf_ref, sc_ref, sh_ref, w2_ref, b2_ref,
              wa1_ref, ba1_ref, sw1_ref, gf2_ref, st_ref):
    p = pl.program_id(0)
    xh = pe1_ref[...] * sc_ref[...] + sh_ref[...]
    pe = jnp.where(xh >= 0, xh, 0.2 * xh)
    pe2 = lax.dot_general(pe, w2_ref[...], (((1,), (0,)), ((), ())),
                          preferred_element_type=jnp.float32) + b2_ref[...]
    gf = gf_ref[...]
    qk = kf_ref[...].reshape(MBLK, 1, C) - gf.reshape(MBLK, K, C)
    t = qk.reshape(MBLK * K, C) + pe2
    sw1 = lax.dot_general(t, wa1_ref[...], (((1,), (0,)), ((), ())),
                          preferred_element_type=jnp.float32) + ba1_ref[...]
    sw1_ref[...] = sw1
    gf2_ref[...] = gf + pe2

    @pl.when(p == 0)
    def _():
        st_ref[...] = jnp.zeros((8, C), jnp.float32)

    st_ref[0:1, :] += jnp.sum(sw1, axis=0, keepdims=True)
    st_ref[1:2, :] += jnp.sum(sw1 * sw1, axis=0, keepdims=True)


def _pass3b(pe1, group_f, key_f, sc1, sh1, w2t, b2, wa1t, ba1):
    grid = (B * M // MBLK,)
    r = MBLK * K
    return pl.pallas_call(
        _p3b_body,
        grid=grid,
        in_specs=[
            pl.BlockSpec((r, C), lambda p: (p, 0)),
            pl.BlockSpec((r, C), lambda p: (p + NKEY // r, 0)),
            pl.BlockSpec((MBLK, C), lambda p: (p, 0)),
            pl.BlockSpec((1, C), lambda p: (0, 0)),
            pl.BlockSpec((1, C), lambda p: (0, 0)),
            pl.BlockSpec((C, C), lambda p: (0, 0)),
            pl.BlockSpec((1, C), lambda p: (0, 0)),
            pl.BlockSpec((C, C), lambda p: (0, 0)),
            pl.BlockSpec((1, C), lambda p: (0, 0)),
        ],
        out_specs=[
            pl.BlockSpec((r, C), lambda p: (p, 0)),
            pl.BlockSpec((r, C), lambda p: (p, 0)),
            pl.BlockSpec((8, C), lambda p: (0, 0)),
        ],
        out_shape=[
            jax.ShapeDtypeStruct((NROW, C), jnp.float32),
            jax.ShapeDtypeStruct((NROW, C), jnp.float32),
            jax.ShapeDtypeStruct((8, C), jnp.float32),
        ],
    )(pe1, group_f, key_f, sc1, sh1, w2t, b2, wa1t, ba1)


def _p3c_body(sw1_ref, gf2_ref, gv_ref, sc_ref, sh_ref, wf_ref, bf_ref,
              wp_ref, bp_ref, nf_ref, np_ref):
    xh = sw1_ref[...] * sc_ref[...] + sh_ref[...]
    sw = jnp.where(xh >= 0, xh, 0.2 * xh)
    a = lax.dot_general(sw, wf_ref[...], (((1,), (0,)), ((), ())),
                        preferred_element_type=jnp.float32) + bf_ref[...]
    pch = lax.dot_general(sw, wp_ref[...], (((1,), (0,)), ((), ())),
                          preferred_element_type=jnp.float32) + bp_ref[...]

    a3 = a.reshape(MBLK, K, C)
    am = jnp.max(a3, axis=1, keepdims=True)
    ae = jnp.exp(a3 - am)
    aw = ae / jnp.sum(ae, axis=1, keepdims=True)
    nf_ref[...] = jnp.sum(aw * gf2_ref[...].reshape(MBLK, K, C), axis=1)

    p3 = pch.reshape(MBLK, K, 16)
    pm = jnp.max(p3, axis=1, keepdims=True)
    pe = jnp.exp(p3 - pm)
    pw = pe / jnp.sum(pe, axis=1, keepdims=True)
    np_ref[...] = jnp.sum(pw * gv_ref[...].reshape(MBLK, K, 16), axis=1)


def _pass3c(sw1, gf2, group_v, sc2, sh2, wa2tf, ba2f, wa2tp, ba2p):
    grid = (B * M // MBLK,)
    r = MBLK * K
    return pl.pallas_call(
        _p3c_body,
        grid=grid,
        in_specs=[
            pl.BlockSpec((r, C), lambda p: (p, 0)),
            pl.BlockSpec((r, C), lambda p: (p, 0)),
            pl.BlockSpec((r, 16), lambda p: (p + NKEY // r, 0)),
            pl.BlockSpec((1, C), lambda p: (0, 0)),
            pl.BlockSpec((1, C), lambda p: (0, 0)),
            pl.BlockSpec((C, C), lambda p: (0, 0)),
            pl.BlockSpec((1, C), lambda p: (0, 0)),
            pl.BlockSpec((C, 16), lambda p: (0, 0)),
            pl.BlockSpec((1, 16), lambda p: (0, 0)),
        ],
        out_specs=[
            pl.BlockSpec((MBLK, C), lambda p: (p, 0)),
            pl.BlockSpec((MBLK, 16), lambda p: (p, 0)),
        ],
        out_shape=[
            jax.ShapeDtypeStruct((NKEY, C), jnp.float32),
            jax.ShapeDtypeStruct((NKEY, 16), jnp.float32),
        ],
    )(sw1, gf2, group_v, sc2, sh2, wa2tf, ba2f, wa2tp, ba2p)


def _bn_affine(s_row, ss_row, gamma, beta, n):
    mean = s_row / n
    var = jnp.maximum(ss_row / n - mean * mean, 0.0)
    scale = gamma / jnp.sqrt(var + 1e-5)
    shift = beta - mean * scale
    return scale.reshape(1, C), shift.reshape(1, C)


@jax.jit
def kernel(vertices, feature_map, W1, b1, g1, be1, W2, b2, Wa1, ba1, ga,
           bea, Wa2, ba2):
    f32 = jnp.float32
    table_f = feature_map.transpose(0, 2, 1).reshape(B * N, C)
    table_v = jnp.zeros((B * N, 16), f32).at[:, :3].set(
        vertices.transpose(0, 2, 1).reshape(B * N, 3))

    kp4 = _fps(vertices)
    keys = kp4.transpose(1, 0, 2)
    kp_flat = keys[:, :, 3].astype(jnp.int32).reshape(NKEY)

    knn = _knn(vertices, keys)
    idx_all = jnp.concatenate([kp_flat, knn.reshape(NROW)])

    out_f, out_v = _sc_gather(table_f, table_v, idx_all)
    key_f = out_f[:NKEY]
    key_v = out_v[:NKEY]

    w1t = jnp.zeros((16, C), f32).at[:3].set(W1.T)
    pe1, st1 = _pass3a(out_v, key_v, w1t, b1.reshape(1, C))
    sc1, sh1 = _bn_affine(st1[0], st1[1], g1, be1, float(NROW))

    sw1, gf2, st2 = _pass3b(pe1, out_f, key_f, sc1, sh1, W2.T,
                            b2.reshape(1, C), Wa1.T, ba1.reshape(1, C))
    sc2, sh2 = _bn_affine(st2[0], st2[1], ga, bea, float(NROW))

    wa2tf = Wa2[3:].T
    ba2f = ba2[3:].reshape(1, C)
    wa2tp = jnp.zeros((C, 16), f32).at[:, :3].set(Wa2[:3].T)
    ba2p = jnp.zeros((1, 16), f32).at[0, :3].set(ba2[:3])
    nf, npt = _pass3c(sw1, gf2, out_v, sc2, sh2, wa2tf, ba2f, wa2tp, ba2p)

    new_feat = nf.reshape(B, M, C).transpose(0, 2, 1)
    new_point = npt[:, :3].reshape(B, M, 3).transpose(0, 2, 1)
    return new_point, new_feat
